# Initial kernel scaffold; baseline (speedup 1.0000x reference)
import jax
import jax.numpy as jnp
from jax.experimental import pallas as pl


def kernel(input, op, hidden_stack, pos):
    raise NotImplementedError("write your pallas kernel here")


# speedup vs baseline: 1.1605x; 1.1605x over previous
import functools

import jax
import jax.numpy as jnp
from jax import lax
from jax.experimental import pallas as pl
from jax.experimental.pallas import tpu as pltpu
from jax.experimental.pallas import tpu_sc as plsc

_SEQ = 512
_B = 2048
_H = 64

_info = plsc.get_sparse_core_info()
_NC, _NS, _L = _info.num_cores, _info.num_subcores, _info.num_lanes
_NW = _NC * _NS
_BPW = _B // _NW


def _sc_body(stack_hbm, inp_hbm, pos_hbm, op_hbm, out_hbm,
             pos_v, op_v, idx_v, rows_v, inp_v, sem):
    wid = lax.axis_index("s") * _NC + lax.axis_index("c")
    base = wid * _BPW
    pltpu.sync_copy(pos_hbm.at[pl.ds(base, _BPW)], pos_v)
    pltpu.sync_copy(op_hbm.at[pl.ds(base, _BPW)], op_v)
    pltpu.sync_copy(inp_hbm.at[pl.ds(base, _BPW)], inp_v)
    lane = lax.iota(jnp.int32, _L)
    for g in range(_BPW // _L):
        opg = op_v[pl.ds(g * _L, _L)]
        posg = pos_v[pl.ds(g * _L, _L)]
        new_pos = posg + opg - 1
        idx_v[pl.ds(g * _L, _L)] = new_pos * _B + (base + g * _L) + lane
    pltpu.async_copy(stack_hbm.at[idx_v], rows_v, sem).wait()
    for g in range(_BPW // _L):
        opg = op_v[pl.ds(g * _L, _L)]
        for l in range(_L):
            b = g * _L + l
            @pl.when(opg[l] == 2)
            def _():
                for j in range(_H // _L):
                    rows_v[b, pl.ds(j * _L, _L)] = inp_v[b, pl.ds(j * _L, _L)]
    pltpu.sync_copy(rows_v, out_hbm.at[pl.ds(base, _BPW)])


_sc_call = pl.kernel(
    _sc_body,
    out_type=jax.ShapeDtypeStruct((_B, _H), jnp.float32),
    mesh=plsc.VectorSubcoreMesh(core_axis_name="c", subcore_axis_name="s"),
    scratch_types=[
        pltpu.VMEM((_BPW,), jnp.int32),
        pltpu.VMEM((_BPW,), jnp.int32),
        pltpu.VMEM((_BPW,), jnp.int32),
        pltpu.VMEM((_BPW, _H), jnp.float32),
        pltpu.VMEM((_BPW, _H), jnp.float32),
        pltpu.SemaphoreType.DMA,
    ],
    compiler_params=pltpu.CompilerParams(use_tc_tiling_on_sc=False),
)


@jax.jit
def kernel(input, op, hidden_stack, pos):
    stack_flat = hidden_stack.reshape(-1, _H)
    return _sc_call(stack_flat, input,
                    pos.astype(jnp.int32), op.astype(jnp.int32))


# speedup vs baseline: 13.9685x; 12.0362x over previous
import jax
import jax.numpy as jnp
from jax import lax
from jax.experimental import pallas as pl
from jax.experimental.pallas import tpu as pltpu
from jax.experimental.pallas import tpu_sc as plsc

_SEQ2 = 514
_B = 2048
_H = 64

_info = plsc.get_sparse_core_info()
_NC, _NS, _L = _info.num_cores, _info.num_subcores, _info.num_lanes
_NW = _NC * _NS
_BT = 128
_NG = _BT // _L
_HH = _H // 2
_NQ = 32
_QI = _BT


def _sc_body(stack_hbm, inp_hbm, pos_hbm, op_hbm, out_hbm,
             pos_v, op_v, idx_v, dst_v, out3_v, inp_v, sem_a, sem_b):
    wid = lax.axis_index("s") * _NC + lax.axis_index("c")
    c = wid // 2
    half = wid % 2
    base = pl.multiple_of(c * _BT, _BT)
    ho = half * _HH
    pltpu.sync_copy(pos_hbm.at[pl.ds(base, _BT)], pos_v)
    pltpu.sync_copy(op_hbm.at[pl.ds(base, _BT)], op_v)
    pltpu.sync_copy(inp_hbm.at[pl.ds(ho, _HH), pl.ds(base, _BT)], inp_v)

    lane = lax.iota(jnp.int32, _L)
    op_chunks = [op_v[pl.ds(t * _L, _L)] for t in range(_NG)]
    pos_chunks = [pos_v[pl.ds(t * _L, _L)] for t in range(_NG)]
    sels = [o == 2 for o in op_chunks]
    w_chunks = [(p + o - 1) * _H + ho
                for p, o in zip(pos_chunks, op_chunks)]
    jvs = [lane + (hloc * _L) for hloc in range(8)]

    for t in range(_NG):
        for hq in range(4):
            q = t * 4 + hq
            for hloc in range(8):
                idx_v[pl.ds((q * 8 + hloc) * _L, _L)] = (
                    w_chunks[t] + (hq * 8 + hloc))

    def issue(q, parity, sem):
        pltpu.async_copy(
            stack_hbm.at[idx_v.at[pl.ds(q * _QI, _QI)], pl.ds(base, _BT)],
            dst_v.at[pl.ds(parity * _QI, _QI)], sem)

    def drain(parity, sem):
        pltpu.make_async_copy(
            stack_hbm.at[pl.ds(0, _QI), pl.ds(0, _BT)],
            dst_v.at[pl.ds(parity * _QI, _QI)], sem).wait()

    issue(0, 0, sem_a)

    def step(q, carry):
        parity = lax.rem(q, 2)
        nxt = q + 1
        nparity = lax.rem(nxt, 2)

        @pl.when(jnp.logical_and(nxt < _NQ, nparity == 0))
        def _():
            issue(nxt, nparity, sem_a)

        @pl.when(jnp.logical_and(nxt < _NQ, nparity == 1))
        def _():
            issue(nxt, nparity, sem_b)

        @pl.when(parity == 0)
        def _():
            drain(parity, sem_a)

        @pl.when(parity == 1)
        def _():
            drain(parity, sem_b)

        t = q // 4
        hq = lax.rem(q, 4)
        tds = t * _L
        sel = op_v[pl.ds(tds, _L)] == 2
        ev = lane + tds
        for hloc in range(8):
            hr = hq * 8 + hloc
            g = plsc.load_gather(dst_v, [jvs[hloc] + parity * _QI, ev])
            new = inp_v[hr, pl.ds(tds, _L)]
            out3_v[hr, pl.ds(tds, _L)] = jnp.where(sel, new, g)
        return carry

    lax.fori_loop(0, _NQ, step, 0)

    pltpu.sync_copy(out3_v, out_hbm.at[pl.ds(ho, _HH), pl.ds(base, _BT)])


_sc_call = pl.kernel(
    _sc_body,
    out_type=jax.ShapeDtypeStruct((_H, _B), jnp.float32),
    mesh=plsc.VectorSubcoreMesh(core_axis_name="c", subcore_axis_name="s"),
    scratch_types=[
        pltpu.VMEM((_BT,), jnp.int32),
        pltpu.VMEM((_BT,), jnp.int32),
        pltpu.VMEM((_NQ * _QI,), jnp.int32),
        pltpu.VMEM((2 * _QI, _BT), jnp.float32),
        pltpu.VMEM((_HH, _BT), jnp.float32),
        pltpu.VMEM((_HH, _BT), jnp.float32),
        pltpu.SemaphoreType.DMA,
        pltpu.SemaphoreType.DMA,
    ],
    compiler_params=pltpu.CompilerParams(needs_layout_passes=False),
)


@jax.jit
def kernel(input, op, hidden_stack, pos):
    stack_v = jnp.transpose(hidden_stack, (0, 2, 1)).reshape(_SEQ2 * _H, _B)
    inp2 = jnp.transpose(input, (1, 0))
    out = _sc_call(stack_v, inp2,
                   pos.astype(jnp.int32), op.astype(jnp.int32))
    return jnp.transpose(out, (1, 0))


# speedup vs baseline: 14.6192x; 1.0466x over previous
import jax
import jax.numpy as jnp
from jax import lax
from jax.experimental import pallas as pl
from jax.experimental.pallas import tpu as pltpu
from jax.experimental.pallas import tpu_sc as plsc

_SEQ2 = 514
_B = 2048
_H = 64

_info = plsc.get_sparse_core_info()
_NC, _NS, _L = _info.num_cores, _info.num_subcores, _info.num_lanes
_NW = _NC * _NS
_BT = 128
_NG = _BT // _L
_HH = _H // 2
_NQ = 16
_QI = 256


def _sc_body(stack_hbm, inp_hbm, pos_hbm, op_hbm, out_hbm,
             pos_v, op_v, idx_v, dst_v, out3_v, inp_v, sem_a, sem_b):
    wid = lax.axis_index("s") * _NC + lax.axis_index("c")
    c = wid // 2
    half = wid % 2
    base = pl.multiple_of(c * _BT, _BT)
    ho = half * _HH
    pltpu.sync_copy(pos_hbm.at[pl.ds(base, _BT)], pos_v)
    pltpu.sync_copy(op_hbm.at[pl.ds(base, _BT)], op_v)
    pltpu.sync_copy(inp_hbm.at[pl.ds(ho, _HH), pl.ds(base, _BT)], inp_v)

    lane = lax.iota(jnp.int32, _L)
    op_chunks = [op_v[pl.ds(t * _L, _L)] for t in range(_NG)]
    pos_chunks = [pos_v[pl.ds(t * _L, _L)] for t in range(_NG)]
    sels = [o == 2 for o in op_chunks]
    w_chunks = [(p + o - 1) * _H + ho
                for p, o in zip(pos_chunks, op_chunks)]
    jvs = [lane + (hloc * _L) for hloc in range(16)]

    for t in range(_NG):
        for hh in range(2):
            q = t * 2 + hh
            for hloc in range(16):
                idx_v[pl.ds((q * 16 + hloc) * _L, _L)] = (
                    w_chunks[t] + (hh * 16 + hloc))

    def issue(q, parity, sem):
        pltpu.async_copy(
            stack_hbm.at[idx_v.at[pl.ds(q * _QI, _QI)], pl.ds(base, _BT)],
            dst_v.at[pl.ds(parity * _QI, _QI)], sem)

    def drain(parity, sem):
        pltpu.make_async_copy(
            stack_hbm.at[pl.ds(0, _QI), pl.ds(0, _BT)],
            dst_v.at[pl.ds(parity * _QI, _QI)], sem).wait()

    issue(0, 0, sem_a)

    def step(q, carry):
        parity = lax.rem(q, 2)
        nxt = q + 1
        nparity = lax.rem(nxt, 2)

        @pl.when(jnp.logical_and(nxt < _NQ, nparity == 0))
        def _():
            issue(nxt, nparity, sem_a)

        @pl.when(jnp.logical_and(nxt < _NQ, nparity == 1))
        def _():
            issue(nxt, nparity, sem_b)

        @pl.when(parity == 0)
        def _():
            drain(parity, sem_a)

        @pl.when(parity == 1)
        def _():
            drain(parity, sem_b)

        t = q // 2
        hh = lax.rem(q, 2)
        tds = t * _L
        sel = op_v[pl.ds(tds, _L)] == 2
        ev = lane + tds
        for hloc in range(16):
            hr = hh * 16 + hloc
            g = plsc.load_gather(dst_v, [jvs[hloc] + parity * _QI, ev])
            new = inp_v[hr, pl.ds(tds, _L)]
            out3_v[hr, pl.ds(tds, _L)] = jnp.where(sel, new, g)
        return carry

    lax.fori_loop(0, _NQ, step, 0)

    pltpu.sync_copy(out3_v, out_hbm.at[pl.ds(ho, _HH), pl.ds(base, _BT)])


_sc_call = pl.kernel(
    _sc_body,
    out_type=jax.ShapeDtypeStruct((_H, _B), jnp.float32),
    mesh=plsc.VectorSubcoreMesh(core_axis_name="c", subcore_axis_name="s"),
    scratch_types=[
        pltpu.VMEM((_BT,), jnp.int32),
        pltpu.VMEM((_BT,), jnp.int32),
        pltpu.VMEM((_NQ * _QI,), jnp.int32),
        pltpu.VMEM((2 * _QI, _BT), jnp.float32),
        pltpu.VMEM((_HH, _BT), jnp.float32),
        pltpu.VMEM((_HH, _BT), jnp.float32),
        pltpu.SemaphoreType.DMA,
        pltpu.SemaphoreType.DMA,
    ],
    compiler_params=pltpu.CompilerParams(needs_layout_passes=False),
)


@jax.jit
def kernel(input, op, hidden_stack, pos):
    stack_v = jnp.transpose(hidden_stack, (0, 2, 1)).reshape(_SEQ2 * _H, _B)
    inp2 = jnp.transpose(input, (1, 0))
    out = _sc_call(stack_v, inp2,
                   pos.astype(jnp.int32), op.astype(jnp.int32))
    return jnp.transpose(out, (1, 0))


# speedup vs baseline: 14.6213x; 1.0001x over previous
import jax
import jax.numpy as jnp
from jax import lax
from jax.experimental import pallas as pl
from jax.experimental.pallas import tpu as pltpu
from jax.experimental.pallas import tpu_sc as plsc

_SEQ2 = 514
_B = 2048
_H = 64

_info = plsc.get_sparse_core_info()
_NC, _NS, _L = _info.num_cores, _info.num_subcores, _info.num_lanes
_NW = _NC * _NS
_BT = 128
_NG = _BT // _L
_HH = _H // 2
_NQ = 16
_QI = 256


def _sc_body(stack_hbm, inp_hbm, pos_hbm, op_hbm, out_hbm,
             pos_v, op_v, idx_v, dst_v, out3_v, inp_v, sem_a, sem_b):
    wid = lax.axis_index("s") * _NC + lax.axis_index("c")
    c = wid // 2
    half = wid % 2
    base = pl.multiple_of(c * _BT, _BT)
    ho = half * _HH
    pltpu.sync_copy(pos_hbm.at[pl.ds(base, _BT)], pos_v)
    pltpu.sync_copy(op_hbm.at[pl.ds(base, _BT)], op_v)
    pltpu.sync_copy(inp_hbm.at[pl.ds(ho, _HH), pl.ds(base, _BT)], inp_v)

    lane = lax.iota(jnp.int32, _L)
    op_chunks = [op_v[pl.ds(t * _L, _L)] for t in range(_NG)]
    pos_chunks = [pos_v[pl.ds(t * _L, _L)] for t in range(_NG)]
    sels = [o == 2 for o in op_chunks]
    w_chunks = [(p + o - 1) * _H + ho
                for p, o in zip(pos_chunks, op_chunks)]
    jvs = [lane + (hloc * _L) for hloc in range(16)]

    for t in range(_NG):
        for hh in range(2):
            q = t * 2 + hh
            for hloc in range(16):
                idx_v[pl.ds((q * 16 + hloc) * _L, _L)] = (
                    w_chunks[t] + (hh * 16 + hloc))

    def issue(q, parity, sem):
        pltpu.async_copy(
            stack_hbm.at[idx_v.at[pl.ds(q * _QI, _QI)], pl.ds(base, _BT)],
            dst_v.at[pl.ds(parity * _QI, _QI)], sem)

    def drain(parity, sem):
        pltpu.make_async_copy(
            stack_hbm.at[pl.ds(0, _QI), pl.ds(0, _BT)],
            dst_v.at[pl.ds(parity * _QI, _QI)], sem).wait()

    issue(0, 0, sem_a)

    def step(q, carry):
        parity = lax.rem(q, 2)
        nxt = q + 1
        nparity = lax.rem(nxt, 2)

        @pl.when(jnp.logical_and(nxt < _NQ, nparity == 0))
        def _():
            issue(nxt, nparity, sem_a)

        @pl.when(jnp.logical_and(nxt < _NQ, nparity == 1))
        def _():
            issue(nxt, nparity, sem_b)

        @pl.when(parity == 0)
        def _():
            drain(parity, sem_a)

        @pl.when(parity == 1)
        def _():
            drain(parity, sem_b)

        t = q // 2
        hh = lax.rem(q, 2)
        tds = t * _L
        sel = op_v[pl.ds(tds, _L)] == 2
        ev = lane + tds
        for hloc in range(16):
            hr = hh * 16 + hloc
            g = plsc.load_gather(dst_v, [jvs[hloc] + parity * _QI, ev])
            new = inp_v[hr, pl.ds(tds, _L)]
            out3_v[hr, pl.ds(tds, _L)] = jnp.where(sel, new, g)
        return carry

    lax.fori_loop(0, _NQ, step, 0)

    pltpu.sync_copy(out3_v, out_hbm.at[pl.ds(ho, _HH), pl.ds(base, _BT)])


_sc_call = pl.kernel(
    _sc_body,
    out_type=jax.ShapeDtypeStruct((_H, _B), jnp.float32),
    mesh=plsc.VectorSubcoreMesh(core_axis_name="c", subcore_axis_name="s"),
    scratch_types=[
        pltpu.VMEM((_BT,), jnp.int32),
        pltpu.VMEM((_BT,), jnp.int32),
        pltpu.VMEM((_NQ * _QI,), jnp.int32),
        pltpu.VMEM((2 * _QI, _BT), jnp.float32),
        pltpu.VMEM((_HH, _BT), jnp.float32),
        pltpu.VMEM((_HH, _BT), jnp.float32),
        pltpu.SemaphoreType.DMA,
        pltpu.SemaphoreType.DMA,
    ],
    compiler_params=pltpu.CompilerParams(needs_layout_passes=False,
                                         skip_device_barrier=True),
)


@jax.jit
def kernel(input, op, hidden_stack, pos):
    stack_v = jnp.transpose(hidden_stack, (0, 2, 1)).reshape(_SEQ2 * _H, _B)
    inp2 = jnp.transpose(input, (1, 0))
    out = _sc_call(stack_v, inp2,
                   pos.astype(jnp.int32), op.astype(jnp.int32))
    return jnp.transpose(out, (1, 0))
